# baseline (speedup 1.0000x reference)
import functools

import jax
import jax.numpy as jnp
from jax import lax
from jax.experimental import pallas as pl
from jax.experimental.pallas import tpu as pltpu
from jax.experimental.pallas import tpu_sc as plsc

NUM_CORES = 2
NUM_SUBCORES = 16
NUM_WORKERS = NUM_CORES * NUM_SUBCORES
CHUNK = 128
K = 4


def _gather_rows(table, idx):
    n = idx.shape[0]
    d = table.shape[1]
    per_w = n // NUM_WORKERS
    group = K * CHUNK
    n_groups = per_w // group
    n_pairs = n_groups // 2
    assert per_w * NUM_WORKERS == n
    assert n_pairs * 2 * group == per_w

    mesh = plsc.VectorSubcoreMesh(core_axis_name="c", subcore_axis_name="s")

    @functools.partial(
        pl.kernel,
        out_type=jax.ShapeDtypeStruct((n, d), jnp.float32),
        mesh=mesh,
        scratch_types=[
            pltpu.VMEM((per_w,), jnp.int32),
            pltpu.VMEM((2, group, d), jnp.float32),
            pltpu.SemaphoreType.DMA,
            pltpu.SemaphoreType.DMA,
            pltpu.SemaphoreType.DMA,
        ],
        compiler_params=pltpu.CompilerParams(use_tc_tiling_on_sc=False),
    )
    def body(table_hbm, idx_hbm, out_hbm, idx_v, rows_v, gsem0, gsem1, osem):
        wid = lax.axis_index("s") * NUM_CORES + lax.axis_index("c")
        base = wid * per_w
        pltpu.sync_copy(idx_hbm.at[pl.ds(base, per_w)], idx_v)

        def gather_desc(goff, j, half):
            return pltpu.make_async_copy(
                table_hbm.at[idx_v.at[pl.ds(goff + j * CHUNK, CHUNK)]],
                rows_v.at[half, pl.ds(j * CHUNK, CHUNK)],
                gsem0 if half == 0 else gsem1,
            )

        def issue_group(goff, half):
            for j in range(K):
                gather_desc(goff, j, half).start()

        def drain_group(half):
            for j in range(K):
                gather_desc(0, j, half).wait()

        def out_desc(goff, half):
            return pltpu.make_async_copy(
                rows_v.at[half],
                out_hbm.at[pl.ds(base + goff, group)],
                osem,
            )

        issue_group(0, 0)

        def loop_body(p, carry):
            for h in range(2):
                g = 2 * p + h
                goff = g * group
                nxt = 1 - h

                @pl.when(g + 1 < n_groups)
                def _():
                    @pl.when(g >= 1)
                    def _():
                        out_desc(0, nxt).wait()
                    issue_group(goff + group, nxt)

                drain_group(h)
                out_desc(goff, h).start()
            return carry

        lax.fori_loop(0, n_pairs, loop_body, 0)
        out_desc(0, 0).wait()
        out_desc(0, 1).wait()

    return body(table, idx)


def kernel(go, emb_weights):
    b, h = go.shape
    d = emb_weights.shape[1]
    idx = jnp.maximum(go.reshape(-1).astype(jnp.int32), 0)
    one = (idx[0] >= jnp.int32(0)).astype(jnp.float32)
    table = emb_weights * one
    out = _gather_rows(table, idx) * one
    return out.reshape(b, h, d)


# speedup vs baseline: 1.6188x; 1.6188x over previous
import functools

import jax
import jax.numpy as jnp
from jax import lax
from jax.experimental import pallas as pl
from jax.experimental.pallas import tpu as pltpu
from jax.experimental.pallas import tpu_sc as plsc

NUM_CORES = 2
NUM_SUBCORES = 16
NUM_WORKERS = NUM_CORES * NUM_SUBCORES
CHUNK = 128
HB = 2 * CHUNK
LANES = 16


def _gather_lmajor(t128, idxT, batch, hist, d):
    nq, qw = t128.shape
    bw = batch // NUM_WORKERS
    n_steps = hist * (bw // HB)
    halves = bw // HB
    assert bw % HB == 0 and qw == 4 * d and d % LANES == 0

    mesh = plsc.VectorSubcoreMesh(core_axis_name="c", subcore_axis_name="s")

    @functools.partial(
        pl.kernel,
        out_type=jax.ShapeDtypeStruct((hist, d, batch), jnp.float32),
        mesh=mesh,
        scratch_types=[
            pltpu.VMEM((2, HB), jnp.int32),
            pltpu.VMEM((2, HB), jnp.int32),
            pltpu.VMEM((2, HB), jnp.int32),
            pltpu.VMEM((2, HB, qw), jnp.float32),
            pltpu.VMEM((2, d, HB), jnp.float32),
            pltpu.SemaphoreType.DMA,
            pltpu.SemaphoreType.DMA,
            pltpu.SemaphoreType.DMA,
        ],
        compiler_params=pltpu.CompilerParams(
            use_tc_tiling_on_sc=False, needs_layout_passes=False),
    )
    def body(t_hbm, idx_hbm, out_hbm, idx_v, q_v, m_v, quad_v, oblk_v,
             gsem0, gsem1, osem):
        wid = lax.axis_index("s") * NUM_CORES + lax.axis_index("c")
        b0 = wid * bw

        def stage(s, buf):
            l = s // halves
            off = l * batch + b0 + (s % halves) * HB
            pltpu.sync_copy(idx_hbm.at[pl.ds(off, HB)], idx_v.at[buf])
            for c16 in range(HB // LANES):
                sl = pl.ds(c16 * LANES, LANES)
                r = idx_v[buf, sl]
                q_v[buf, sl] = lax.shift_right_logical(r, 2)
                m_v[buf, sl] = lax.shift_left(lax.bitwise_and(r, 3), 5)

        def gather_descs(buf):
            sem = gsem0 if buf == 0 else gsem1
            return [
                pltpu.make_async_copy(
                    t_hbm.at[q_v.at[buf, pl.ds(j * CHUNK, CHUNK)]],
                    quad_v.at[buf, pl.ds(j * CHUNK, CHUNK)],
                    sem,
                )
                for j in range(HB // CHUNK)
            ]

        def out_desc(s, buf):
            l = s // halves
            boff = b0 + (s % halves) * HB
            return pltpu.make_async_copy(
                oblk_v.at[buf],
                out_hbm.at[l, :, pl.ds(boff, HB)],
                osem,
            )

        def extract(buf):
            for c16 in range(HB // LANES):
                rowv = lax.iota(jnp.int32, LANES) + c16 * LANES
                colb = m_v[buf, pl.ds(c16 * LANES, LANES)]
                for c in range(d):
                    vals = plsc.load_gather(
                        quad_v.at[buf], [rowv, colb + c])
                    oblk_v[buf, c, pl.ds(c16 * LANES, LANES)] = vals

        stage(0, 0)
        for dsc in gather_descs(0):
            dsc.start()

        def step_body(s, buf):
            @pl.when(s + 1 < n_steps)
            def _():
                stage(s + 1, 1 - buf)
                for dsc in gather_descs(1 - buf):
                    dsc.start()

            for dsc in gather_descs(buf):
                dsc.wait()

            @pl.when(s >= 2)
            def _():
                out_desc(s, buf).wait()

            extract(buf)
            out_desc(s, buf).start()

        def pair_body(p, carry):
            step_body(2 * p, 0)
            step_body(2 * p + 1, 1)
            return carry

        lax.fori_loop(0, n_steps // 2, pair_body, 0)
        out_desc(0, 0).wait()
        out_desc(0, 1).wait()

    return body(t128, idxT)


def kernel(go, emb_weights):
    b, h = go.shape
    v, d = emb_weights.shape
    idxT = jnp.maximum(go.T.reshape(-1).astype(jnp.int32), 0)
    one = (idxT[0] >= jnp.int32(0)).astype(jnp.float32)
    t128 = (emb_weights * one).reshape(v * d // 128, 128)
    out2 = _gather_lmajor(t128, idxT, b, h, d)
    return jnp.transpose(out2, (2, 0, 1))


# speedup vs baseline: 1.6826x; 1.0395x over previous
import functools

import jax
import jax.numpy as jnp
from jax import lax
from jax.experimental import pallas as pl
from jax.experimental.pallas import tpu as pltpu
from jax.experimental.pallas import tpu_sc as plsc

NUM_CORES = 2
NUM_SUBCORES = 16
NUM_WORKERS = NUM_CORES * NUM_SUBCORES
CHUNK = 128
LANES = 16


def _gather_lmajor(table, idxT, batch, hist):
    v, d = table.shape
    bw = batch // NUM_WORKERS
    assert bw % CHUNK == 0 and d % LANES == 0

    mesh = plsc.VectorSubcoreMesh(core_axis_name="c", subcore_axis_name="s")

    @functools.partial(
        pl.kernel,
        out_type=jax.ShapeDtypeStruct((hist, d, batch), jnp.float32),
        mesh=mesh,
        scratch_types=[
            pltpu.VMEM((hist * bw,), jnp.int32),
            pltpu.VMEM((2, bw, d), jnp.float32),
            pltpu.VMEM((2, d, bw), jnp.float32),
            pltpu.SemaphoreType.DMA,
            pltpu.SemaphoreType.DMA,
            pltpu.SemaphoreType.DMA,
            pltpu.SemaphoreType.DMA,
        ],
        compiler_params=pltpu.CompilerParams(
            use_tc_tiling_on_sc=False, needs_layout_passes=False),
    )
    def body(t_hbm, idx_hbm, out_hbm, idx_v, g_v, oblk_v,
             isem, gsem0, gsem1, osem):
        wid = lax.axis_index("s") * NUM_CORES + lax.axis_index("c")
        b0 = wid * bw

        for l in range(hist):
            pltpu.async_copy(
                idx_hbm.at[pl.ds(l * batch + b0, bw)],
                idx_v.at[pl.ds(l * bw, bw)],
                isem,
            )
        pltpu.make_async_copy(idx_hbm.at[pl.ds(0, hist * bw)], idx_v, isem
                              ).wait()

        def gather_descs(l, buf):
            sem = gsem0 if buf == 0 else gsem1
            return [
                pltpu.make_async_copy(
                    t_hbm.at[idx_v.at[pl.ds(l * bw + j * CHUNK, CHUNK)]],
                    g_v.at[buf, pl.ds(j * CHUNK, CHUNK)],
                    sem,
                )
                for j in range(bw // CHUNK)
            ]

        def out_desc(l, buf):
            return pltpu.make_async_copy(
                oblk_v.at[buf],
                out_hbm.at[l, :, pl.ds(b0, bw)],
                osem,
            )

        def transpose(buf):
            for c16 in range(bw // LANES):
                rowv = lax.iota(jnp.int32, LANES) + c16 * LANES
                for c in range(d):
                    colv = jnp.full((LANES,), c, jnp.int32)
                    vals = plsc.load_gather(g_v.at[buf], [rowv, colv])
                    oblk_v[buf, c, pl.ds(c16 * LANES, LANES)] = vals

        for dsc in gather_descs(0, 0):
            dsc.start()

        def step(l, buf):
            @pl.when(l + 1 < hist)
            def _():
                for dsc in gather_descs(l + 1, 1 - buf):
                    dsc.start()

            for dsc in gather_descs(0, buf):
                dsc.wait()

            @pl.when(l >= 2)
            def _():
                out_desc(0, buf).wait()

            transpose(buf)
            out_desc(l, buf).start()

        def pair_body(p, carry):
            step(2 * p, 0)
            step(2 * p + 1, 1)
            return carry

        lax.fori_loop(0, hist // 2, pair_body, 0)
        out_desc(0, 0).wait()
        out_desc(0, 1).wait()

    return body(table, idxT)


def kernel(go, emb_weights):
    b, h = go.shape
    idxT = jnp.maximum(go.T.reshape(-1).astype(jnp.int32), 0)
    out2 = _gather_lmajor(emb_weights, idxT, b, h)
    return jnp.transpose(out2, (2, 0, 1))


# speedup vs baseline: 2.1590x; 1.2831x over previous
import functools

import jax
import jax.numpy as jnp
from jax import lax
from jax.experimental import pallas as pl
from jax.experimental.pallas import tpu as pltpu
from jax.experimental.pallas import tpu_sc as plsc

NUM_CORES = 2
NUM_SUBCORES = 16
NUM_WORKERS = NUM_CORES * NUM_SUBCORES
CHUNK = 128
LANES = 16


def _gather_lmajor(table, idxT, batch, hist):
    v, d = table.shape
    bw = batch // NUM_WORKERS
    assert bw % CHUNK == 0 and d % LANES == 0

    mesh = plsc.VectorSubcoreMesh(core_axis_name="c", subcore_axis_name="s")

    @functools.partial(
        pl.kernel,
        out_type=jax.ShapeDtypeStruct((hist, d, batch), jnp.float32),
        mesh=mesh,
        scratch_types=[
            pltpu.VMEM((hist * bw,), jnp.int32),
            pltpu.VMEM((2, bw, d), jnp.float32),
            pltpu.VMEM((2, d, bw), jnp.float32),
            pltpu.SemaphoreType.DMA,
            pltpu.SemaphoreType.DMA,
            pltpu.SemaphoreType.DMA,
            pltpu.SemaphoreType.DMA,
        ],
        compiler_params=pltpu.CompilerParams(
            use_tc_tiling_on_sc=False, needs_layout_passes=False),
    )
    def body(t_hbm, idx_hbm, out_hbm, idx_v, g_v, oblk_v,
             isem, gsem0, gsem1, osem):
        wid = lax.axis_index("s") * NUM_CORES + lax.axis_index("c")
        b0 = wid * bw

        for l in range(hist):
            pltpu.async_copy(
                idx_hbm.at[pl.ds(l * batch + b0, bw)],
                idx_v.at[pl.ds(l * bw, bw)],
                isem,
            )
        pltpu.make_async_copy(idx_hbm.at[pl.ds(0, hist * bw)], idx_v, isem
                              ).wait()

        def gather_descs(l, buf):
            sem = gsem0 if buf == 0 else gsem1
            return [
                pltpu.make_async_copy(
                    t_hbm.at[idx_v.at[pl.ds(l * bw + j * CHUNK, CHUNK)]],
                    g_v.at[buf, pl.ds(j * CHUNK, CHUNK)],
                    sem,
                )
                for j in range(bw // CHUNK)
            ]

        def out_desc(l, buf):
            return pltpu.make_async_copy(
                oblk_v.at[buf],
                out_hbm.at[l, :, pl.ds(b0, bw)],
                osem,
            )

        def transpose(buf):
            @plsc.parallel_loop(0, bw // LANES, 1, unroll=4)
            def _(c16):
                rowv = lax.iota(jnp.int32, LANES) + c16 * LANES
                for c in range(d):
                    colv = jnp.full((LANES,), c, jnp.int32)
                    vals = plsc.load_gather(g_v.at[buf], [rowv, colv])
                    oblk_v[buf, c, pl.ds(c16 * LANES, LANES)] = vals

        for dsc in gather_descs(0, 0):
            dsc.start()

        def step(l, buf):
            @pl.when(l + 1 < hist)
            def _():
                for dsc in gather_descs(l + 1, 1 - buf):
                    dsc.start()

            for dsc in gather_descs(0, buf):
                dsc.wait()

            @pl.when(l >= 2)
            def _():
                out_desc(0, buf).wait()

            transpose(buf)
            out_desc(l, buf).start()

        def pair_body(p, carry):
            step(2 * p, 0)
            step(2 * p + 1, 1)
            return carry

        lax.fori_loop(0, hist // 2, pair_body, 0)
        out_desc(0, 0).wait()
        out_desc(0, 1).wait()

    return body(table, idxT)


def kernel(go, emb_weights):
    b, h = go.shape
    idxT = jnp.maximum(go.T.reshape(-1).astype(jnp.int32), 0)
    out2 = _gather_lmajor(emb_weights, idxT, b, h)
    return jnp.transpose(out2, (2, 0, 1))
